# baseline (speedup 1.0000x reference)
import jax
import jax.numpy as jnp
from jax import lax
from jax.experimental import pallas as pl
from jax.experimental.pallas import tpu as pltpu


def _matvec_kernel(x_ref, w_ref, b_ref, o_ref):
    xb = x_ref[...].astype(jnp.bfloat16)
    wb = w_ref[...].astype(jnp.bfloat16)
    acc = lax.dot_general(
        xb, wb,
        dimension_numbers=(((1,), (1,)), ((), ())),
        preferred_element_type=jnp.float32,
    )
    o_ref[...] = acc + b_ref[...]


def kernel(x, weight, bias):
    B, F = x.shape
    C, F_w = weight.shape
    assert F == F_w and bias.shape == (C,)

    tm = 2048
    while B % tm != 0 and tm > 8:
        tm //= 2
    n_tiles = pl.cdiv(B, tm)
    B_pad = n_tiles * tm
    if B_pad != B:
        x = jnp.pad(x, ((0, B_pad - B), (0, 0)))

    out = pl.pallas_call(
        _matvec_kernel,
        out_shape=jax.ShapeDtypeStruct((B_pad, C), jnp.float32),
        grid=(n_tiles,),
        in_specs=[
            pl.BlockSpec((tm, F), lambda i: (i, 0)),
            pl.BlockSpec((C, F), lambda i: (0, 0)),
            pl.BlockSpec((1, C), lambda i: (0, 0)),
        ],
        out_specs=pl.BlockSpec((tm, C), lambda i: (i, 0)),
        compiler_params=pltpu.CompilerParams(
            dimension_semantics=("parallel",),
        ),
        cost_estimate=pl.CostEstimate(
            flops=2 * B_pad * F * C,
            transcendentals=0,
            bytes_accessed=4 * (B_pad * F + B_pad * C + C * F),
        ),
    )(x, weight, bias.reshape(1, C))
    return out[:B]


# speedup vs baseline: 1.0552x; 1.0552x over previous
import jax
import jax.numpy as jnp
from jax import lax
from jax.experimental import pallas as pl
from jax.experimental.pallas import tpu as pltpu


def _matvec_kernel(x_ref, w_ref, b_ref, o_ref):
    xb = x_ref[...].astype(jnp.bfloat16)
    wb = w_ref[...].astype(jnp.bfloat16)
    acc = lax.dot_general(
        xb, wb,
        dimension_numbers=(((1,), (1,)), ((), ())),
        preferred_element_type=jnp.float32,
    )
    o_ref[...] = acc + b_ref[...]


def kernel(x, weight, bias):
    B, F = x.shape
    C, F_w = weight.shape
    assert F == F_w and bias.shape == (C,)

    tm = 1024
    while B % tm != 0 and tm > 8:
        tm //= 2
    n_tiles = pl.cdiv(B, tm)
    B_pad = n_tiles * tm
    if B_pad != B:
        x = jnp.pad(x, ((0, B_pad - B), (0, 0)))

    out = pl.pallas_call(
        _matvec_kernel,
        out_shape=jax.ShapeDtypeStruct((B_pad, C), jnp.float32),
        grid=(n_tiles,),
        in_specs=[
            pl.BlockSpec((tm, F), lambda i: (i, 0)),
            pl.BlockSpec((C, F), lambda i: (0, 0)),
            pl.BlockSpec((1, C), lambda i: (0, 0)),
        ],
        out_specs=pl.BlockSpec((tm, C), lambda i: (i, 0)),
        compiler_params=pltpu.CompilerParams(
            dimension_semantics=("parallel",),
        ),
        cost_estimate=pl.CostEstimate(
            flops=2 * B_pad * F * C,
            transcendentals=0,
            bytes_accessed=4 * (B_pad * F + B_pad * C + C * F),
        ),
    )(x, weight, bias.reshape(1, C))
    return out[:B]


# speedup vs baseline: 1.3215x; 1.2524x over previous
import jax
import jax.numpy as jnp
from jax import lax
from jax.experimental import pallas as pl
from jax.experimental.pallas import tpu as pltpu


def _matvec_t_kernel(x_ref, w_ref, b_ref, o_ref):
    xb = x_ref[...].astype(jnp.bfloat16)
    wb = w_ref[...].astype(jnp.bfloat16)
    acc = lax.dot_general(
        wb, xb,
        dimension_numbers=(((1,), (1,)), ((), ())),
        preferred_element_type=jnp.float32,
    )
    o_ref[...] = acc + b_ref[...].T


def kernel(x, weight, bias):
    B, F = x.shape
    C, F_w = weight.shape
    assert F == F_w and bias.shape == (C,)

    tm = 1024
    while B % tm != 0 and tm > 8:
        tm //= 2
    n_tiles = pl.cdiv(B, tm)
    B_pad = n_tiles * tm
    if B_pad != B:
        x = jnp.pad(x, ((0, B_pad - B), (0, 0)))

    out_t = pl.pallas_call(
        _matvec_t_kernel,
        out_shape=jax.ShapeDtypeStruct((C, B_pad), jnp.float32),
        grid=(n_tiles,),
        in_specs=[
            pl.BlockSpec((tm, F), lambda i: (i, 0)),
            pl.BlockSpec((C, F), lambda i: (0, 0)),
            pl.BlockSpec((1, C), lambda i: (0, 0)),
        ],
        out_specs=pl.BlockSpec((C, tm), lambda i: (0, i)),
        compiler_params=pltpu.CompilerParams(
            dimension_semantics=("parallel",),
        ),
        cost_estimate=pl.CostEstimate(
            flops=2 * B_pad * F * C,
            transcendentals=0,
            bytes_accessed=4 * (B_pad * F + B_pad * C + C * F),
        ),
    )(x, weight, bias.reshape(1, C))
    return out_t[:, :B].T
